# baseline (speedup 1.0000x reference)
import functools

import jax
import jax.numpy as jnp
from jax import lax
from jax.experimental import pallas as pl
from jax.experimental.pallas import tpu as pltpu
from jax.experimental.pallas import tpu_sc as plsc


def _lane_allsum(v):
    idx = lax.iota(jnp.int32, 16)
    for sh in (8, 4, 2, 1):
        v = v + v.at[idx ^ sh].get(mode="promise_in_bounds")
    return v


def _round_bf16(v):
    u = lax.bitcast_convert_type(v, jnp.int32)
    u = (u + 0x7FFF + (lax.shift_right_logical(u, 16) & 1)) & (-65536)
    return lax.bitcast_convert_type(u, jnp.float32)


L = 16
NB = 6
NT = 13
RT = 6
NO = 90
XR = 0
WB = 80 * 80
AX = WB + 80 * NB * L


def _body(pack_hbm, out_hbm, xs_v, wb_v, ax_v, est_v, sem):
    tid = lax.axis_index("s")

    @pl.when(tid < NT)
    def _work():
        cp1 = pltpu.async_copy(
            pack_hbm.at[pl.ds(XR + (RT * tid + 1) * 80, RT * 80)], xs_v, sem)
        cp2 = pltpu.async_copy(pack_hbm.at[pl.ds(WB, 80 * NB * L)], wb_v, sem)
        cp3 = pltpu.async_copy(
            pack_hbm.at[pl.ds(AX, 2 * NB * L + 80)], ax_v, sem)
        cp1.wait()
        cp2.wait()
        cp3.wait()

        X10 = 2 * NB * L
        nrows = RT + 1
        accs = [[jnp.zeros((L,), jnp.float32) for _ in range(NB)]
                for _ in range(nrows)]
        for kb in range(5):
            xblk = [xs_v[pl.ds(r * 80 + kb * L, L)] for r in range(RT)]
            xblk.append(ax_v[pl.ds(X10 + kb * L, L)])
            for l in range(L):
                k = kb * L + l
                wv = [wb_v[pl.ds((k * NB + b) * L, L)] for b in range(NB)]
                for r in range(nrows):
                    c = xblk[r][l]
                    for b in range(NB):
                        accs[r][b] = accs[r][b] + c * wv[b]

        hb = [[_round_bf16(accs[r][b]) for b in range(NB)]
              for r in range(nrows)]

        sacc = jnp.zeros((L,), jnp.float32)
        for b in range(NB):
            sacc = sacc + hb[RT][b] * ax_v[pl.ds(b * L, L)]
        s_vec = _lane_allsum(sacc)

        lane = lax.iota(jnp.int32, L)
        res = jnp.zeros((L,), jnp.float32)
        for r in range(RT):
            tacc = jnp.zeros((L,), jnp.float32)
            for b in range(NB):
                tacc = tacc + hb[r][b] * ax_v[pl.ds(NB * L + b * L, L)]
            t = _lane_allsum(tacc) + s_vec
            e = jnp.where(t >= 0.0, t, 0.2 * t)
            res = jnp.where(lane == r, e, res)
        est_v[...] = res
        pltpu.sync_copy(est_v, out_hbm.at[pl.ds(tid * L, L)])


@functools.partial(
    pl.kernel,
    out_type=jax.ShapeDtypeStruct((NT * L,), jnp.float32),
    mesh=plsc.VectorSubcoreMesh(core_axis_name="c", subcore_axis_name="s",
                                num_cores=1),
    scratch_types=[
        pltpu.VMEM((RT * 80,), jnp.float32),
        pltpu.VMEM((80 * NB * L,), jnp.float32),
        pltpu.VMEM((2 * NB * L + 80,), jnp.float32),
        pltpu.VMEM((L,), jnp.float32),
        pltpu.SemaphoreType.DMA,
    ],
    compiler_params=pltpu.CompilerParams(needs_layout_passes=False),
    name="similarity_sc",
)
def _similarity_sc(pack_hbm, out_hbm, *scratch):
    _body(pack_hbm, out_hbm, *scratch)


def kernel(chicago_region_representations, W_lin, W_attn):
    f32 = jnp.float32
    x = jnp.asarray(chicago_region_representations, f32)
    xbf = _round_bf16(x)
    wbf = _round_bf16(jnp.asarray(W_lin, f32))
    abf = _round_bf16(jnp.asarray(W_attn, f32))[0]
    xrows = jnp.zeros((80, 80), f32).at[0].set(xbf[10]).at[1:78].set(xbf[:77])
    wb = jnp.zeros((80, NB * L), f32).at[:, :NO].set(wbf.T)
    a1 = jnp.zeros((NB * L,), f32).at[:NO].set(abf[:NO])
    a2 = jnp.zeros((NB * L,), f32).at[:NO].set(abf[NO:])
    pack = jnp.concatenate([
        xrows.reshape(-1), wb.reshape(-1), a1, a2, xbf[10],
    ])
    e = _similarity_sc(pack)
    return e.reshape(NT, L)[:, :RT].reshape(-1)[:77].reshape(77, 1)


# speedup vs baseline: 1.0002x; 1.0002x over previous
import functools

import jax
import jax.numpy as jnp
from jax import lax
from jax.experimental import pallas as pl
from jax.experimental.pallas import tpu as pltpu
from jax.experimental.pallas import tpu_sc as plsc


def _lane_allsum(v):
    idx = lax.iota(jnp.int32, 16)
    for sh in (8, 4, 2, 1):
        v = v + v.at[idx ^ sh].get(mode="promise_in_bounds")
    return v


def _round_bf16(v):
    u = lax.bitcast_convert_type(v, jnp.int32)
    u = (u + 0x7FFF + (lax.shift_right_logical(u, 16) & 1)) & (-65536)
    return lax.bitcast_convert_type(u, jnp.float32)


L = 16
NB = 6
NT = 13
RT = 6
NO = 90
XR = 0
WB = 80 * 80
AX = WB + 80 * (NB // 2) * L


def _body(pack_hbm, out_hbm, xs_v, wb_v, ax_v, est_v, sem):
    tid = lax.axis_index("s")

    @pl.when(tid < NT)
    def _work():
        cp1 = pltpu.async_copy(
            pack_hbm.at[pl.ds(XR + (RT * tid + 1) * 80, RT * 80)], xs_v, sem)
        cp2 = pltpu.async_copy(
            pack_hbm.at[pl.ds(WB, 80 * (NB // 2) * L)], wb_v, sem)
        cp3 = pltpu.async_copy(
            pack_hbm.at[pl.ds(AX, 2 * NB * L + 80)], ax_v, sem)
        cp1.wait()
        cp2.wait()
        cp3.wait()

        X10 = 2 * NB * L
        nrows = RT + 1
        accs = [[jnp.zeros((L,), jnp.float32) for _ in range(NB)]
                for _ in range(nrows)]
        for kb in range(5):
            xblk = [xs_v[pl.ds(r * 80 + kb * L, L)] for r in range(RT)]
            xblk.append(ax_v[pl.ds(X10 + kb * L, L)])
            for l in range(L):
                k = kb * L + l
                wv = []
                for p in range(NB // 2):
                    pair = plsc.bitcast(
                        wb_v[pl.ds((k * (NB // 2) + p) * L, L)], jnp.bfloat16)
                    wa, wbv = plsc.unpack(pair,
                                          format=plsc.PackFormat.INTERLEAVED)
                    wv.extend((wa, wbv))
                for r in range(nrows):
                    c = xblk[r][l]
                    for b in range(NB):
                        accs[r][b] = accs[r][b] + c * wv[b]

        hb = [[_round_bf16(accs[r][b]) for b in range(NB)]
              for r in range(nrows)]

        sacc = jnp.zeros((L,), jnp.float32)
        for b in range(NB):
            sacc = sacc + hb[RT][b] * ax_v[pl.ds(b * L, L)]
        s_vec = _lane_allsum(sacc)

        lane = lax.iota(jnp.int32, L)
        res = jnp.zeros((L,), jnp.float32)
        for r in range(RT):
            tacc = jnp.zeros((L,), jnp.float32)
            for b in range(NB):
                tacc = tacc + hb[r][b] * ax_v[pl.ds(NB * L + b * L, L)]
            t = _lane_allsum(tacc) + s_vec
            e = jnp.where(t >= 0.0, t, 0.2 * t)
            res = jnp.where(lane == r, e, res)
        est_v[...] = res
        pltpu.sync_copy(est_v, out_hbm.at[pl.ds(tid * L, L)])


@functools.partial(
    pl.kernel,
    out_type=jax.ShapeDtypeStruct((NT * L,), jnp.float32),
    mesh=plsc.VectorSubcoreMesh(core_axis_name="c", subcore_axis_name="s",
                                num_cores=1),
    scratch_types=[
        pltpu.VMEM((RT * 80,), jnp.float32),
        pltpu.VMEM((80 * (NB // 2) * L,), jnp.float32),
        pltpu.VMEM((2 * NB * L + 80,), jnp.float32),
        pltpu.VMEM((L,), jnp.float32),
        pltpu.SemaphoreType.DMA,
    ],
    compiler_params=pltpu.CompilerParams(needs_layout_passes=False),
    name="similarity_sc",
)
def _similarity_sc(pack_hbm, out_hbm, *scratch):
    _body(pack_hbm, out_hbm, *scratch)


def kernel(chicago_region_representations, W_lin, W_attn):
    f32 = jnp.float32
    x = jnp.asarray(chicago_region_representations, f32)
    xbf = _round_bf16(x)
    wbf = _round_bf16(jnp.asarray(W_lin, f32))
    abf = _round_bf16(jnp.asarray(W_attn, f32))[0]
    xrows = jnp.zeros((80, 80), f32).at[0].set(xbf[10]).at[1:78].set(xbf[:77])
    wb96 = jnp.zeros((80, NB * L), f32).at[:, :NO].set(wbf.T)
    wpairs = wb96.reshape(80, NB // 2, 2, L).transpose(0, 1, 3, 2)
    wb = lax.bitcast_convert_type(
        wpairs.astype(jnp.bfloat16), f32)
    a1 = jnp.zeros((NB * L,), f32).at[:NO].set(abf[:NO])
    a2 = jnp.zeros((NB * L,), f32).at[:NO].set(abf[NO:])
    pack = jnp.concatenate([
        xrows.reshape(-1), wb.reshape(-1), a1, a2, xbf[10],
    ])
    e = _similarity_sc(pack)
    return e.reshape(NT, L)[:, :RT].reshape(-1)[:77].reshape(77, 1)


# speedup vs baseline: 1.1431x; 1.1429x over previous
import functools

import jax
import jax.numpy as jnp
from jax import lax
from jax.experimental import pallas as pl
from jax.experimental.pallas import tpu as pltpu
from jax.experimental.pallas import tpu_sc as plsc


def _lane_allsum(v):
    idx = lax.iota(jnp.int32, 16)
    for sh in (8, 4, 2, 1):
        v = v + v.at[idx ^ sh].get(mode="promise_in_bounds")
    return v


def _round_bf16(v):
    u = lax.bitcast_convert_type(v, jnp.int32)
    u = (u + 0x7FFF + (lax.shift_right_logical(u, 16) & 1)) & (-65536)
    return lax.bitcast_convert_type(u, jnp.float32)


L = 16
NB = 6
NT = 13
RT = 6
NO = 90
XR = 0
WB = 80 * 80
AX = WB + 80 * (NB // 2) * L


def _body(pack_hbm, out_hbm, xs_v, wb_v, ax_v, est_v, sem):
    tid = lax.axis_index("s")

    @pl.when(tid < NT)
    def _work():
        cp1 = pltpu.async_copy(
            pack_hbm.at[pl.ds(XR + (RT * tid + 1) * 80, RT * 80)], xs_v, sem)
        cp2 = pltpu.async_copy(
            pack_hbm.at[pl.ds(WB, 80 * (NB // 2) * L)], wb_v, sem)
        cp3 = pltpu.async_copy(
            pack_hbm.at[pl.ds(AX, 2 * NB * L + 80)], ax_v, sem)
        cp1.wait()
        cp2.wait()
        cp3.wait()

        X10 = 2 * NB * L
        nrows = RT + 1

        def kb_step(kb, carry):
            accs = [list(row) for row in carry]
            xblk = [xs_v[pl.ds(r * 80 + kb * L, L)] for r in range(RT)]
            xblk.append(ax_v[pl.ds(X10 + kb * L, L)])
            for l in range(L):
                k = kb * L + l
                wv = []
                for p in range(NB // 2):
                    pair = plsc.bitcast(
                        wb_v[pl.ds((k * (NB // 2) + p) * L, L)], jnp.bfloat16)
                    wa, wbv = plsc.unpack(pair,
                                          format=plsc.PackFormat.INTERLEAVED)
                    wv.extend((wa, wbv))
                for r in range(nrows):
                    c = xblk[r][l]
                    for b in range(NB):
                        accs[r][b] = accs[r][b] + c * wv[b]
            return tuple(tuple(row) for row in accs)

        zero = jnp.zeros((L,), jnp.float32)
        accs = lax.fori_loop(
            0, 5, kb_step,
            tuple(tuple(zero for _ in range(NB)) for _ in range(nrows)))

        hb = [[_round_bf16(accs[r][b]) for b in range(NB)]
              for r in range(nrows)]

        sacc = jnp.zeros((L,), jnp.float32)
        for b in range(NB):
            sacc = sacc + hb[RT][b] * ax_v[pl.ds(b * L, L)]
        s_vec = _lane_allsum(sacc)

        lane = lax.iota(jnp.int32, L)
        res = jnp.zeros((L,), jnp.float32)
        for r in range(RT):
            tacc = jnp.zeros((L,), jnp.float32)
            for b in range(NB):
                tacc = tacc + hb[r][b] * ax_v[pl.ds(NB * L + b * L, L)]
            t = _lane_allsum(tacc) + s_vec
            e = jnp.where(t >= 0.0, t, 0.2 * t)
            res = jnp.where(lane == r, e, res)
        est_v[...] = res
        pltpu.sync_copy(est_v, out_hbm.at[pl.ds(tid * L, L)])


@functools.partial(
    pl.kernel,
    out_type=jax.ShapeDtypeStruct((NT * L,), jnp.float32),
    mesh=plsc.VectorSubcoreMesh(core_axis_name="c", subcore_axis_name="s",
                                num_cores=1),
    scratch_types=[
        pltpu.VMEM((RT * 80,), jnp.float32),
        pltpu.VMEM((80 * (NB // 2) * L,), jnp.float32),
        pltpu.VMEM((2 * NB * L + 80,), jnp.float32),
        pltpu.VMEM((L,), jnp.float32),
        pltpu.SemaphoreType.DMA,
    ],
    compiler_params=pltpu.CompilerParams(needs_layout_passes=False),
    name="similarity_sc",
)
def _similarity_sc(pack_hbm, out_hbm, *scratch):
    _body(pack_hbm, out_hbm, *scratch)


def kernel(chicago_region_representations, W_lin, W_attn):
    f32 = jnp.float32
    x = jnp.asarray(chicago_region_representations, f32)
    xbf = _round_bf16(x)
    wbf = _round_bf16(jnp.asarray(W_lin, f32))
    abf = _round_bf16(jnp.asarray(W_attn, f32))[0]
    xrows = jnp.zeros((80, 80), f32).at[0].set(xbf[10]).at[1:78].set(xbf[:77])
    wb96 = jnp.zeros((80, NB * L), f32).at[:, :NO].set(wbf.T)
    wpairs = wb96.reshape(80, NB // 2, 2, L).transpose(0, 1, 3, 2)
    wb = lax.bitcast_convert_type(
        wpairs.astype(jnp.bfloat16), f32)
    a1 = jnp.zeros((NB * L,), f32).at[:NO].set(abf[:NO])
    a2 = jnp.zeros((NB * L,), f32).at[:NO].set(abf[NO:])
    pack = jnp.concatenate([
        xrows.reshape(-1), wb.reshape(-1), a1, a2, xbf[10],
    ])
    e = _similarity_sc(pack)
    return e.reshape(NT, L)[:, :RT].reshape(-1)[:77].reshape(77, 1)
